# baseline (speedup 1.0000x reference)
import functools
import numpy as np
import jax
import jax.numpy as jnp
from jax import lax
from jax.experimental import pallas as pl
from jax.experimental.pallas import tpu as pltpu
from jax.experimental.pallas import tpu_sc as plsc

_LS = [0, 1, 2, 3, 4, 5, 6]
_MAXM = max(_LS)
_DIM = sum(2 * l + 1 for l in _LS)


def _real_basis_u(l):
    d = 2 * l + 1
    U = np.zeros((d, d), dtype=np.complex128)
    U[l, l] = 1.0
    for m in range(1, l + 1):
        U[l + m, l + m] = ((-1.0) ** m) / np.sqrt(2.0)
        U[l + m, l - m] = 1.0 / np.sqrt(2.0)
        U[l - m, l + m] = -1j * ((-1.0) ** m) / np.sqrt(2.0)
        U[l - m, l - m] = 1j / np.sqrt(2.0)
    return U


def _j_matrix(l, theta=-np.pi / 2):
    d = 2 * l + 1
    ms = np.arange(-l, l + 1)
    Lp = np.zeros((d, d))
    for i in range(d - 1):
        mm = ms[i]
        Lp[i + 1, i] = np.sqrt(l * (l + 1) - mm * (mm + 1))
    Lx = (Lp + Lp.T) / 2.0
    w, V = np.linalg.eigh(Lx)
    Dc = (V * np.exp(-1j * theta * w)) @ V.conj().T
    U = _real_basis_u(l)
    return np.real(U @ Dc @ U.conj().T)


def _clean(J):
    J = np.where(np.abs(J) < 1e-6, 0.0, J)
    J = np.where(np.abs(J - 1.0) < 1e-6, 1.0, J)
    J = np.where(np.abs(J + 1.0) < 1e-6, -1.0, J)
    return np.asarray(J, dtype=np.float32)


_J_NP = [_clean(_j_matrix(l)) for l in _LS]


def _z_apply(rows, l, cosv, sinv):
    d = 2 * l + 1
    out = []
    for k in range(d):
        m = k - l
        if m == 0:
            out.append(rows[k])
        elif m > 0:
            out.append(cosv[m] * rows[k] + sinv[m] * rows[2 * l - k])
        else:
            out.append(cosv[-m] * rows[k] - sinv[-m] * rows[2 * l - k])
    return out


def _j_apply(rows, Jm):
    d = Jm.shape[0]
    out = []
    for i in range(d):
        acc = None
        for j in range(d):
            v = float(Jm[i, j])
            if v == 0.0:
                continue
            term = rows[j] if v == 1.0 else v * rows[j]
            acc = term if acc is None else acc + term
        out.append(acc)
    return out


_NW = 32
_LANES = 16


def _tec_body(x_hbm, sa_hbm, sb_hbm, sg_hbm, o_hbm,
              xv, ov, scv, sem0, sem1, per_w):
    C = x_hbm.shape[-1]
    nchunk = C // _LANES
    cidx = lax.axis_index("c")
    sidx = lax.axis_index("s")
    wid = sidx * 2 + cidx
    base = wid * per_w

    pltpu.sync_copy(sa_hbm.at[:, :, pl.ds(base, per_w)], scv.at[0])
    pltpu.sync_copy(sb_hbm.at[:, :, pl.ds(base, per_w)], scv.at[1])
    pltpu.sync_copy(sg_hbm.at[:, :, pl.ds(base, per_w)], scv.at[2])

    def in_copy(g, buf, sem):
        return pltpu.make_async_copy(
            x_hbm.at[:, pl.ds(base + g, 1), :], xv.at[buf], sem)

    in_copy(0, 0, sem0).start()
    in_copy(1, 1, sem1).start()

    def process(g, buf, sem):
        in_copy(g, buf, sem).wait()
        g16 = (g // _LANES) * _LANES
        lane = jnp.full((_LANES,), g - g16, jnp.int32)

        def splat(a, m, which):
            vec = scv[a, m, which, pl.ds(g16, _LANES)]
            return vec.at[lane].get(mode="promise_in_bounds")

        coef = []
        for a in range(3):
            cosv, sinv = {}, {}
            for m in range(1, _MAXM + 1):
                cosv[m] = splat(a, m, 1)
                sinv[m] = splat(a, m, 0)
            coef.append((cosv, sinv))
        (ca, sa), (cb, sb), (cg, sg) = coef

        def chunk(cc, _):
            sl = pl.ds(cc * _LANES, _LANES)
            for l in _LS:
                d = 2 * l + 1
                off = l * l
                rows = [xv[buf, off + k, 0, sl] for k in range(d)]
                t = _z_apply(rows, l, cg, sg)
                t = _j_apply(t, _J_NP[l].T)
                t = _z_apply(t, l, cb, sb)
                t = _j_apply(t, _J_NP[l])
                t = _z_apply(t, l, ca, sa)
                for k in range(d):
                    ov[off + k, 0, sl] = t[k]
            return 0

        lax.fori_loop(0, nchunk, chunk, 0, unroll=False)
        pltpu.sync_copy(ov, o_hbm.at[:, pl.ds(base + g, 1), :])

    def outer(i2, _):
        g0 = i2 * 2
        for b, sem in ((0, sem0), (1, sem1)):
            g = g0 + b
            process(g, b, sem)

            @pl.when(g + 2 < per_w)
            def _():
                in_copy(g + 2, b, sem).start()
        return 0

    lax.fori_loop(0, per_w // 2, outer, 0, unroll=False)


@jax.jit
def kernel(input, sincos_alpha, sincos_beta, sincos_gamma):
    B, dim, C = input.shape
    per_w = B // _NW
    xt = jnp.transpose(input, (1, 0, 2))
    scs = [jnp.transpose(s, (2, 1, 0))
           for s in (sincos_alpha, sincos_beta, sincos_gamma)]
    mesh = plsc.VectorSubcoreMesh(core_axis_name="c", subcore_axis_name="s")
    f = functools.partial(
        pl.kernel,
        mesh=mesh,
        out_type=jax.ShapeDtypeStruct((dim, B, C), input.dtype),
        scratch_types=[
            pltpu.VMEM((2, dim, 1, C), jnp.float32),
            pltpu.VMEM((dim, 1, C), jnp.float32),
            pltpu.VMEM((3, _MAXM + 1, 2, per_w), jnp.float32),
            pltpu.SemaphoreType.DMA,
            pltpu.SemaphoreType.DMA,
        ],
    )(functools.partial(_tec_body, per_w=per_w))
    yt = f(xt, *scs)
    return jnp.transpose(yt, (1, 0, 2))


# speedup vs baseline: 1.0102x; 1.0102x over previous
import functools
import numpy as np
import jax
import jax.numpy as jnp
from jax import lax
from jax.experimental import pallas as pl
from jax.experimental.pallas import tpu as pltpu
from jax.experimental.pallas import tpu_sc as plsc

_LS = [0, 1, 2, 3, 4, 5, 6]
_MAXM = max(_LS)
_DIM = sum(2 * l + 1 for l in _LS)


def _real_basis_u(l):
    d = 2 * l + 1
    U = np.zeros((d, d), dtype=np.complex128)
    U[l, l] = 1.0
    for m in range(1, l + 1):
        U[l + m, l + m] = ((-1.0) ** m) / np.sqrt(2.0)
        U[l + m, l - m] = 1.0 / np.sqrt(2.0)
        U[l - m, l + m] = -1j * ((-1.0) ** m) / np.sqrt(2.0)
        U[l - m, l - m] = 1j / np.sqrt(2.0)
    return U


def _j_matrix(l, theta=-np.pi / 2):
    d = 2 * l + 1
    ms = np.arange(-l, l + 1)
    Lp = np.zeros((d, d))
    for i in range(d - 1):
        mm = ms[i]
        Lp[i + 1, i] = np.sqrt(l * (l + 1) - mm * (mm + 1))
    Lx = (Lp + Lp.T) / 2.0
    w, V = np.linalg.eigh(Lx)
    Dc = (V * np.exp(-1j * theta * w)) @ V.conj().T
    U = _real_basis_u(l)
    return np.real(U @ Dc @ U.conj().T)


def _clean(J):
    J = np.where(np.abs(J) < 1e-6, 0.0, J)
    J = np.where(np.abs(J - 1.0) < 1e-6, 1.0, J)
    J = np.where(np.abs(J + 1.0) < 1e-6, -1.0, J)
    return np.asarray(J, dtype=np.float32)


_J_NP = [_clean(_j_matrix(l)) for l in _LS]


def _z_apply(rows, l, cosv, sinv):
    d = 2 * l + 1
    out = []
    for k in range(d):
        m = k - l
        if m == 0:
            out.append(rows[k])
        elif m > 0:
            out.append(cosv[m] * rows[k] + sinv[m] * rows[2 * l - k])
        else:
            out.append(cosv[-m] * rows[k] - sinv[-m] * rows[2 * l - k])
    return out


def _j_apply(rows, Jm):
    d = Jm.shape[0]
    out = []
    for i in range(d):
        acc = None
        for j in range(d):
            v = float(Jm[i, j])
            if v == 0.0:
                continue
            term = rows[j] if v == 1.0 else v * rows[j]
            acc = term if acc is None else acc + term
        out.append(acc)
    return out


_NW = 32
_LANES = 16


_NB = 2


def _tec_body(x_hbm, sa_hbm, sb_hbm, sg_hbm, o_hbm,
              xv, ov, scv, isem0, isem1, osem0, osem1, per_w):
    C = x_hbm.shape[-1]
    nchunk = C // _LANES
    cidx = lax.axis_index("c")
    sidx = lax.axis_index("s")
    wid = sidx * 2 + cidx
    base = wid * per_w
    ngrp = per_w // _NB

    pltpu.sync_copy(sa_hbm.at[:, :, pl.ds(base, per_w)], scv.at[0])
    pltpu.sync_copy(sb_hbm.at[:, :, pl.ds(base, per_w)], scv.at[1])
    pltpu.sync_copy(sg_hbm.at[:, :, pl.ds(base, per_w)], scv.at[2])

    def in_copy(grp, buf, sem):
        return pltpu.make_async_copy(
            x_hbm.at[:, pl.ds(base + grp * _NB, _NB), :], xv.at[buf], sem)

    def out_copy(grp, buf, sem):
        return pltpu.make_async_copy(
            ov.at[buf], o_hbm.at[:, pl.ds(base + grp * _NB, _NB), :], sem)

    in_copy(0, 0, isem0).start()
    in_copy(1, 1, isem1).start()

    def process(grp, buf, isem, osem):
        in_copy(grp, buf, isem).wait()

        @pl.when(grp >= 2)
        def _():
            out_copy(grp - 2, buf, osem).wait()

        for j in range(_NB):
            g = grp * _NB + j
            g16 = (g // _LANES) * _LANES
            lane = jnp.full((_LANES,), g - g16, jnp.int32)

            def splat(a, m, which, g16=g16, lane=lane):
                vec = scv[a, m, which, pl.ds(g16, _LANES)]
                return vec.at[lane].get(mode="promise_in_bounds")

            coef = []
            for a in range(3):
                cosv, sinv = {}, {}
                for m in range(1, _MAXM + 1):
                    cosv[m] = splat(a, m, 1)
                    sinv[m] = splat(a, m, 0)
                coef.append((cosv, sinv))
            (ca, sa), (cb, sb), (cg, sg) = coef

            def chunk(cc, _, j=j, ca=ca, sa=sa, cb=cb, sb=sb, cg=cg, sg=sg):
                sl = pl.ds(cc * _LANES, _LANES)
                for l in _LS:
                    d = 2 * l + 1
                    off = l * l
                    rows = [xv[buf, off + k, j, sl] for k in range(d)]
                    t = _z_apply(rows, l, cg, sg)
                    t = _j_apply(t, _J_NP[l].T)
                    t = _z_apply(t, l, cb, sb)
                    t = _j_apply(t, _J_NP[l])
                    t = _z_apply(t, l, ca, sa)
                    for k in range(d):
                        ov[buf, off + k, j, sl] = t[k]
                return 0

            lax.fori_loop(0, nchunk, chunk, 0, unroll=False)
        out_copy(grp, buf, osem).start()

    def outer(i2, _):
        g0 = i2 * 2
        process(g0, 0, isem0, osem0)

        @pl.when(g0 + 2 < ngrp)
        def _():
            in_copy(g0 + 2, 0, isem0).start()

        process(g0 + 1, 1, isem1, osem1)

        @pl.when(g0 + 3 < ngrp)
        def _():
            in_copy(g0 + 3, 1, isem1).start()

        return 0

    lax.fori_loop(0, ngrp // 2, outer, 0, unroll=False)
    out_copy(ngrp - 2, 0, osem0).wait()
    out_copy(ngrp - 1, 1, osem1).wait()


@jax.jit
def kernel(input, sincos_alpha, sincos_beta, sincos_gamma):
    B, dim, C = input.shape
    per_w = B // _NW
    xt = jnp.transpose(input, (1, 0, 2))
    scs = [jnp.transpose(s, (2, 1, 0))
           for s in (sincos_alpha, sincos_beta, sincos_gamma)]
    mesh = plsc.VectorSubcoreMesh(core_axis_name="c", subcore_axis_name="s")
    f = functools.partial(
        pl.kernel,
        mesh=mesh,
        out_type=jax.ShapeDtypeStruct((dim, B, C), input.dtype),
        scratch_types=[
            pltpu.VMEM((2, dim, _NB, C), jnp.float32),
            pltpu.VMEM((2, dim, _NB, C), jnp.float32),
            pltpu.VMEM((3, _MAXM + 1, 2, per_w), jnp.float32),
            pltpu.SemaphoreType.DMA,
            pltpu.SemaphoreType.DMA,
            pltpu.SemaphoreType.DMA,
            pltpu.SemaphoreType.DMA,
        ],
    )(functools.partial(_tec_body, per_w=per_w))
    yt = f(xt, *scs)
    return jnp.transpose(yt, (1, 0, 2))


# speedup vs baseline: 1.1592x; 1.1476x over previous
import functools
import numpy as np
import jax
import jax.numpy as jnp
from jax import lax
from jax.experimental import pallas as pl
from jax.experimental.pallas import tpu as pltpu
from jax.experimental.pallas import tpu_sc as plsc

_LS = [0, 1, 2, 3, 4, 5, 6]
_MAXM = max(_LS)
_DIM = sum(2 * l + 1 for l in _LS)


def _real_basis_u(l):
    d = 2 * l + 1
    U = np.zeros((d, d), dtype=np.complex128)
    U[l, l] = 1.0
    for m in range(1, l + 1):
        U[l + m, l + m] = ((-1.0) ** m) / np.sqrt(2.0)
        U[l + m, l - m] = 1.0 / np.sqrt(2.0)
        U[l - m, l + m] = -1j * ((-1.0) ** m) / np.sqrt(2.0)
        U[l - m, l - m] = 1j / np.sqrt(2.0)
    return U


def _j_matrix(l, theta=-np.pi / 2):
    d = 2 * l + 1
    ms = np.arange(-l, l + 1)
    Lp = np.zeros((d, d))
    for i in range(d - 1):
        mm = ms[i]
        Lp[i + 1, i] = np.sqrt(l * (l + 1) - mm * (mm + 1))
    Lx = (Lp + Lp.T) / 2.0
    w, V = np.linalg.eigh(Lx)
    Dc = (V * np.exp(-1j * theta * w)) @ V.conj().T
    U = _real_basis_u(l)
    return np.real(U @ Dc @ U.conj().T)


def _clean(J):
    J = np.where(np.abs(J) < 1e-6, 0.0, J)
    J = np.where(np.abs(J - 1.0) < 1e-6, 1.0, J)
    J = np.where(np.abs(J + 1.0) < 1e-6, -1.0, J)
    return np.asarray(J, dtype=np.float32)


_J_NP = [_clean(_j_matrix(l)) for l in _LS]


def _z_apply(rows, l, cosv, sinv):
    d = 2 * l + 1
    out = []
    for k in range(d):
        m = k - l
        if m == 0:
            out.append(rows[k])
        elif m > 0:
            out.append(cosv[m] * rows[k] + sinv[m] * rows[2 * l - k])
        else:
            out.append(cosv[-m] * rows[k] - sinv[-m] * rows[2 * l - k])
    return out


def _j_apply(rows, Jm):
    d = Jm.shape[0]
    out = []
    for i in range(d):
        acc = None
        for j in range(d):
            v = float(Jm[i, j])
            if v == 0.0:
                continue
            term = rows[j] if v == 1.0 else v * rows[j]
            acc = term if acc is None else acc + term
        out.append(acc)
    return out


_NW = 32
_LANES = 16


_NB = 2


def _tec_body(x_hbm, sa_hbm, sb_hbm, sg_hbm, o_hbm,
              xv, ov, scv, isem0, isem1, osem0, osem1, per_w):
    C = x_hbm.shape[-1]
    nchunk = C // _LANES
    cidx = lax.axis_index("c")
    sidx = lax.axis_index("s")
    wid = sidx * 2 + cidx
    base = wid * per_w
    ngrp = per_w // _NB

    pltpu.sync_copy(sa_hbm.at[:, :, pl.ds(base, per_w)], scv.at[0])
    pltpu.sync_copy(sb_hbm.at[:, :, pl.ds(base, per_w)], scv.at[1])
    pltpu.sync_copy(sg_hbm.at[:, :, pl.ds(base, per_w)], scv.at[2])

    def in_copy(grp, buf, sem):
        return pltpu.make_async_copy(
            x_hbm.at[:, pl.ds(base + grp * _NB, _NB), :], xv.at[buf], sem)

    def out_copy(grp, buf, sem):
        return pltpu.make_async_copy(
            ov.at[buf], o_hbm.at[:, pl.ds(base + grp * _NB, _NB), :], sem)

    in_copy(0, 0, isem0).start()
    in_copy(1, 1, isem1).start()

    def process(grp, buf, isem, osem):
        in_copy(grp, buf, isem).wait()

        @pl.when(grp >= 2)
        def _():
            out_copy(grp - 2, buf, osem).wait()

        for j in range(_NB):
            g = grp * _NB + j
            g16 = (g // _LANES) * _LANES
            lane = jnp.full((_LANES,), g - g16, jnp.int32)

            def splat(a, m, which, g16=g16, lane=lane):
                vec = scv[a, m, which, pl.ds(g16, _LANES)]
                return vec.at[lane].get(mode="promise_in_bounds")

            coef = []
            for a in range(3):
                cosv, sinv = {}, {}
                for m in range(1, _MAXM + 1):
                    cosv[m] = splat(a, m, 1)
                    sinv[m] = splat(a, m, 0)
                coef.append((cosv, sinv))
            (ca, sa), (cb, sb), (cg, sg) = coef

            def chunk(cc, _, j=j, ca=ca, sa=sa, cb=cb, sb=sb, cg=cg, sg=sg):
                sl = pl.ds(cc * _LANES, _LANES)
                for l in _LS:
                    d = 2 * l + 1
                    off = l * l
                    rows = [xv[buf, off + k, j, sl] for k in range(d)]
                    t = _z_apply(rows, l, cg, sg)
                    t = _j_apply(t, _J_NP[l].T)
                    t = _z_apply(t, l, cb, sb)
                    t = _j_apply(t, _J_NP[l])
                    t = _z_apply(t, l, ca, sa)
                    for k in range(d):
                        ov[buf, off + k, j, sl] = t[k]
                return 0

            lax.fori_loop(0, nchunk, chunk, 0, unroll=2)
        out_copy(grp, buf, osem).start()

    def outer(i2, _):
        g0 = i2 * 2
        process(g0, 0, isem0, osem0)

        @pl.when(g0 + 2 < ngrp)
        def _():
            in_copy(g0 + 2, 0, isem0).start()

        process(g0 + 1, 1, isem1, osem1)

        @pl.when(g0 + 3 < ngrp)
        def _():
            in_copy(g0 + 3, 1, isem1).start()

        return 0

    lax.fori_loop(0, ngrp // 2, outer, 0, unroll=False)
    out_copy(ngrp - 2, 0, osem0).wait()
    out_copy(ngrp - 1, 1, osem1).wait()


@jax.jit
def kernel(input, sincos_alpha, sincos_beta, sincos_gamma):
    B, dim, C = input.shape
    per_w = B // _NW
    xt = jnp.transpose(input, (1, 0, 2))
    scs = [jnp.transpose(s, (2, 1, 0))
           for s in (sincos_alpha, sincos_beta, sincos_gamma)]
    mesh = plsc.VectorSubcoreMesh(core_axis_name="c", subcore_axis_name="s")
    f = functools.partial(
        pl.kernel,
        mesh=mesh,
        out_type=jax.ShapeDtypeStruct((dim, B, C), input.dtype),
        scratch_types=[
            pltpu.VMEM((2, dim, _NB, C), jnp.float32),
            pltpu.VMEM((2, dim, _NB, C), jnp.float32),
            pltpu.VMEM((3, _MAXM + 1, 2, per_w), jnp.float32),
            pltpu.SemaphoreType.DMA,
            pltpu.SemaphoreType.DMA,
            pltpu.SemaphoreType.DMA,
            pltpu.SemaphoreType.DMA,
        ],
    )(functools.partial(_tec_body, per_w=per_w))
    yt = f(xt, *scs)
    return jnp.transpose(yt, (1, 0, 2))
